# baseline (device time: 49763 ns/iter reference)
import jax
import jax.numpy as jnp
from jax import lax
from jax.experimental import pallas as pl
from jax.experimental.pallas import tpu as pltpu

N_DEV = 4
BLK = 64
LOG2E = 1.4426950408889634


def kernel(x, Wq, K_ext, V_ext, Wo):
    B, Sq, E = x.shape
    Dm = Wq.shape[1]
    _, Skv_loc, Hq, Dh = K_ext.shape
    HD = Hq * Dh
    ROWS = HD + Hq

    def body(x_ref, wq_ref, k_ref, v_ref, wo_ref, out_ref,
             abuf, send_sems, recv_sems):
        my = lax.axis_index("i")
        left = (my + N_DEV - 1) % N_DEV
        right = (my + 1) % N_DEV

        def rdma(src, dst, s_idx, r_idx, dev):
            return pltpu.make_async_remote_copy(
                src_ref=src, dst_ref=dst,
                send_sem=send_sems.at[s_idx], recv_sem=recv_sems.at[r_idx],
                device_id=(dev,), device_id_type=pl.DeviceIdType.MESH,
            )

        barrier_sem = pltpu.get_barrier_semaphore()
        for nbr in (left, right):
            pl.semaphore_signal(
                barrier_sem, inc=1,
                device_id=(nbr,), device_id_type=pl.DeviceIdType.MESH,
            )
        pl.semaphore_wait(barrier_sem, 2)

        wq = wq_ref[...].astype(jnp.bfloat16)
        QTs = []
        for b in range(B):
            q = lax.dot_general(
                x_ref[b].astype(jnp.bfloat16), wq,
                (((1,), (0,)), ((), ())),
                preferred_element_type=jnp.float32,
            )
            q = (q * (0.125 * LOG2E)).astype(jnp.bfloat16)
            QTs.append(q.T)

        kv_blk = lax.broadcasted_iota(jnp.int32, (Skv_loc, Sq), 0) // BLK
        kv_blk = kv_blk + my * (Skv_loc // BLK)
        q_blk = lax.broadcasted_iota(jnp.int32, (Skv_loc, Sq), 1) // BLK
        maskT = (q_blk == kv_blk) | (kv_blk == 0) | ((q_blk + kv_blk) % 3 == 0)
        bias = jnp.where(maskT, 0.0, -1e30).astype(jnp.float32)

        def partial_attn(b):
            parts = []
            lrows = []
            for h in range(Hq):
                qhT = QTs[b][h * Dh:(h + 1) * Dh, :]
                kh = k_ref[b, :, h, :].astype(jnp.bfloat16)
                sT = lax.dot_general(
                    kh, qhT, (((1,), (0,)), ((), ())),
                    preferred_element_type=jnp.float32,
                )
                pT = jnp.exp2(sT + bias)
                lT = pT.sum(axis=0, keepdims=True)
                vh = v_ref[b, :, h, :].astype(jnp.bfloat16)
                accT = lax.dot_general(
                    vh, pT.astype(jnp.bfloat16),
                    (((0,), (0,)), ((), ())),
                    preferred_element_type=jnp.float32,
                )
                parts.append(accT)
                lrows.append(lT)
            return jnp.concatenate(parts + lrows, axis=0)

        glob = []
        sends = []
        for b in range(B):
            gb = partial_attn(b)
            glob.append(gb)
            abuf[0, b] = gb.astype(jnp.bfloat16)
            to_r = rdma(abuf.at[0, b], abuf.at[1, b], b, b, right)
            to_l = rdma(abuf.at[0, b], abuf.at[2, b], 2 + b, 2 + b, left)
            to_r.start()
            to_l.start()
            sends += [to_r, to_l]

        rdma(abuf.at[0, 0], abuf.at[1, 0], 0, 0, left).wait_recv()
        fwd_r = rdma(abuf.at[1, 0], abuf.at[3, 0], 4, 4, right)
        fwd_r.start()
        rdma(abuf.at[0, 1], abuf.at[2, 1], 3, 3, right).wait_recv()
        fwd_l = rdma(abuf.at[2, 1], abuf.at[3, 1], 5, 5, left)
        fwd_l.start()
        sends += [fwd_r, fwd_l]

        rdma(abuf.at[0, 1], abuf.at[1, 1], 1, 1, left).wait_recv()
        rdma(abuf.at[0, 0], abuf.at[2, 0], 2, 2, right).wait_recv()
        for b in range(B):
            glob[b] = glob[b] + abuf[1, b].astype(jnp.float32) \
                              + abuf[2, b].astype(jnp.float32)

        rdma(abuf.at[1, 0], abuf.at[3, 0], 4, 4, left).wait_recv()
        rdma(abuf.at[2, 1], abuf.at[3, 1], 5, 5, right).wait_recv()
        for b in range(B):
            glob[b] = glob[b] + abuf[3, b].astype(jnp.float32)

        wo = wo_ref[...].astype(jnp.bfloat16)
        for b in range(B):
            accT = glob[b]
            ctx_rows = []
            for h in range(Hq):
                recip = 1.0 / accT[HD + h:HD + h + 1, :]
                ctx_rows.append(accT[h * Dh:(h + 1) * Dh, :] * recip)
            ctxT = jnp.concatenate(ctx_rows, axis=0).astype(jnp.bfloat16)
            out_ref[b] = lax.dot_general(
                ctxT, wo, (((0,), (0,)), ((), ())),
                preferred_element_type=jnp.float32,
            )

        for s in sends:
            s.wait_send()

    return pl.pallas_call(
        body,
        out_shape=jax.ShapeDtypeStruct((B, Sq, E), jnp.float32),
        in_specs=[pl.BlockSpec(memory_space=pltpu.VMEM)] * 5,
        out_specs=pl.BlockSpec(memory_space=pltpu.VMEM),
        scratch_shapes=[
            pltpu.VMEM((4, B, ROWS, Sq), jnp.bfloat16),
            pltpu.SemaphoreType.DMA((6,)),
            pltpu.SemaphoreType.DMA((6,)),
        ],
        compiler_params=pltpu.CompilerParams(
            collective_id=0, vmem_limit_bytes=100 * 1024 * 1024
        ),
    )(x, Wq, K_ext, V_ext, Wo)


# device time: 42557 ns/iter; 1.1693x vs baseline; 1.1693x over previous
import jax
import jax.numpy as jnp
from jax import lax
from jax.experimental import pallas as pl
from jax.experimental.pallas import tpu as pltpu

N_DEV = 4
BLK = 64
LOG2E = 1.4426950408889634


def kernel(x, Wq, K_ext, V_ext, Wo):
    B, Sq, E = x.shape
    Dm = Wq.shape[1]
    _, Skv_loc, Hq, Dh = K_ext.shape
    HD = Hq * Dh
    ROWS = HD + Hq

    K2 = K_ext.reshape(B, Skv_loc, HD).astype(jnp.bfloat16)
    V2 = V_ext.reshape(B, Skv_loc, HD).astype(jnp.bfloat16)

    def body(x_ref, wq_ref, k_ref, v_ref, wo_ref, out_ref,
             abuf, send_sems, recv_sems):
        my = lax.axis_index("i")
        left = (my + N_DEV - 1) % N_DEV
        right = (my + 1) % N_DEV

        def rdma(src, dst, s_idx, r_idx, dev):
            return pltpu.make_async_remote_copy(
                src_ref=src, dst_ref=dst,
                send_sem=send_sems.at[s_idx], recv_sem=recv_sems.at[r_idx],
                device_id=(dev,), device_id_type=pl.DeviceIdType.MESH,
            )

        barrier_sem = pltpu.get_barrier_semaphore()
        for nbr in (left, right):
            pl.semaphore_signal(
                barrier_sem, inc=1,
                device_id=(nbr,), device_id_type=pl.DeviceIdType.MESH,
            )
        pl.semaphore_wait(barrier_sem, 2)

        wq = wq_ref[...].astype(jnp.bfloat16)
        QTs = []
        for b in range(B):
            q = lax.dot_general(
                x_ref[b].astype(jnp.bfloat16), wq,
                (((1,), (0,)), ((), ())),
                preferred_element_type=jnp.float32,
            )
            q = (q * (0.125 * LOG2E)).astype(jnp.bfloat16)
            QTs.append(q.T)

        kv_blk = lax.broadcasted_iota(jnp.int32, (Skv_loc, Sq), 0) // BLK
        kv_blk = kv_blk + my * (Skv_loc // BLK)
        q_blk = lax.broadcasted_iota(jnp.int32, (Skv_loc, Sq), 1) // BLK
        maskT = (q_blk == kv_blk) | (kv_blk == 0) | ((q_blk + kv_blk) % 3 == 0)
        bias = jnp.where(maskT, 0.0, -1e30).astype(jnp.float32)

        def partial_attn(b):
            parts = []
            lrows = []
            for h in range(Hq):
                qhT = QTs[b][h * Dh:(h + 1) * Dh, :]
                kh = k_ref[b, :, h * Dh:(h + 1) * Dh]
                sT = lax.dot_general(
                    kh, qhT, (((1,), (0,)), ((), ())),
                    preferred_element_type=jnp.float32,
                )
                pT = jnp.exp2(sT + bias)
                lT = pT.sum(axis=0, keepdims=True)
                vh = v_ref[b, :, h * Dh:(h + 1) * Dh]
                accT = lax.dot_general(
                    vh, pT.astype(jnp.bfloat16),
                    (((0,), (0,)), ((), ())),
                    preferred_element_type=jnp.float32,
                )
                parts.append(accT)
                lrows.append(lT)
            return jnp.concatenate(parts + lrows, axis=0)

        glob = []
        sends = []
        for b in range(B):
            gb = partial_attn(b)
            glob.append(gb)
            abuf[0, b] = gb.astype(jnp.bfloat16)
            to_r = rdma(abuf.at[0, b], abuf.at[1, b], b, b, right)
            to_l = rdma(abuf.at[0, b], abuf.at[2, b], 2 + b, 2 + b, left)
            to_r.start()
            to_l.start()
            sends += [to_r, to_l]

        rdma(abuf.at[0, 0], abuf.at[1, 0], 0, 0, left).wait_recv()
        fwd_r = rdma(abuf.at[1, 0], abuf.at[3, 0], 4, 4, right)
        fwd_r.start()
        rdma(abuf.at[0, 1], abuf.at[2, 1], 3, 3, right).wait_recv()
        fwd_l = rdma(abuf.at[2, 1], abuf.at[3, 1], 5, 5, left)
        fwd_l.start()
        sends += [fwd_r, fwd_l]

        rdma(abuf.at[0, 1], abuf.at[1, 1], 1, 1, left).wait_recv()
        rdma(abuf.at[0, 0], abuf.at[2, 0], 2, 2, right).wait_recv()
        for b in range(B):
            glob[b] = glob[b] + abuf[1, b].astype(jnp.float32) \
                              + abuf[2, b].astype(jnp.float32)

        rdma(abuf.at[1, 0], abuf.at[3, 0], 4, 4, left).wait_recv()
        rdma(abuf.at[2, 1], abuf.at[3, 1], 5, 5, right).wait_recv()
        for b in range(B):
            glob[b] = glob[b] + abuf[3, b].astype(jnp.float32)

        wo = wo_ref[...].astype(jnp.bfloat16)
        for b in range(B):
            accT = glob[b]
            ctx_rows = []
            for h in range(Hq):
                recip = 1.0 / accT[HD + h:HD + h + 1, :]
                ctx_rows.append(accT[h * Dh:(h + 1) * Dh, :] * recip)
            ctxT = jnp.concatenate(ctx_rows, axis=0).astype(jnp.bfloat16)
            out_ref[b] = lax.dot_general(
                ctxT, wo, (((0,), (0,)), ((), ())),
                preferred_element_type=jnp.float32,
            )

        for s in sends:
            s.wait_send()

    return pl.pallas_call(
        body,
        out_shape=jax.ShapeDtypeStruct((B, Sq, E), jnp.float32),
        in_specs=[pl.BlockSpec(memory_space=pltpu.VMEM)] * 5,
        out_specs=pl.BlockSpec(memory_space=pltpu.VMEM),
        scratch_shapes=[
            pltpu.VMEM((4, B, ROWS, Sq), jnp.bfloat16),
            pltpu.SemaphoreType.DMA((6,)),
            pltpu.SemaphoreType.DMA((6,)),
        ],
        compiler_params=pltpu.CompilerParams(
            collective_id=0, vmem_limit_bytes=100 * 1024 * 1024
        ),
    )(x, Wq, K2, V2, Wo)


# device time: 41967 ns/iter; 1.1858x vs baseline; 1.0141x over previous
import jax
import jax.numpy as jnp
from jax import lax
from jax.experimental import pallas as pl
from jax.experimental.pallas import tpu as pltpu

N_DEV = 4
BLK = 64
LOG2E = 1.4426950408889634


def kernel(x, Wq, K_ext, V_ext, Wo):
    B, Sq, E = x.shape
    Dm = Wq.shape[1]
    _, Skv_loc, Hq, Dh = K_ext.shape
    HD = Hq * Dh
    ROWS = HD + Hq

    K2 = K_ext.reshape(B, Skv_loc, HD).astype(jnp.bfloat16)
    V2 = V_ext.reshape(B, Skv_loc, HD).astype(jnp.bfloat16)

    def body(x_ref, wq_ref, k_ref, v_ref, wo_ref, out_ref,
             abuf, send_sems, recv_sems):
        my = lax.axis_index("i")
        left = (my + N_DEV - 1) % N_DEV
        right = (my + 1) % N_DEV

        def rdma(src, dst, s_idx, r_idx, dev):
            return pltpu.make_async_remote_copy(
                src_ref=src, dst_ref=dst,
                send_sem=send_sems.at[s_idx], recv_sem=recv_sems.at[r_idx],
                device_id=(dev,), device_id_type=pl.DeviceIdType.MESH,
            )

        barrier_sem = pltpu.get_barrier_semaphore()
        for nbr in (left, right):
            pl.semaphore_signal(
                barrier_sem, inc=1,
                device_id=(nbr,), device_id_type=pl.DeviceIdType.MESH,
            )
        pl.semaphore_wait(barrier_sem, 2)

        wq = wq_ref[...].astype(jnp.bfloat16)
        QTs = []
        for b in range(B):
            q = lax.dot_general(
                x_ref[b].astype(jnp.bfloat16), wq,
                (((1,), (0,)), ((), ())),
                preferred_element_type=jnp.float32,
            )
            q = (q * (0.125 * LOG2E)).astype(jnp.bfloat16)
            QTs.append(q.T)

        kv_blk = lax.broadcasted_iota(jnp.int32, (Skv_loc, Sq), 0) // BLK
        kv_blk = kv_blk + my * (Skv_loc // BLK)
        q_blk = lax.broadcasted_iota(jnp.int32, (Skv_loc, Sq), 1) // BLK
        maskT = (q_blk == kv_blk) | (kv_blk == 0) | ((q_blk + kv_blk) % 3 == 0)
        bias = jnp.where(maskT, 0.0, -1e30).astype(jnp.float32)

        def partial_attn(b):
            parts = []
            lrows = []
            for h in range(Hq):
                qhT = QTs[b][h * Dh:(h + 1) * Dh, :]
                kh = k_ref[b, :, h * Dh:(h + 1) * Dh]
                sT = lax.dot_general(
                    kh, qhT, (((1,), (0,)), ((), ())),
                    preferred_element_type=jnp.float32,
                )
                pT = jnp.exp2(sT + bias)
                lT = pT.sum(axis=0, keepdims=True)
                vh = v_ref[b, :, h * Dh:(h + 1) * Dh]
                accT = lax.dot_general(
                    vh, pT.astype(jnp.bfloat16),
                    (((0,), (0,)), ((), ())),
                    preferred_element_type=jnp.float32,
                )
                parts.append(accT)
                lrows.append(lT)
            return jnp.concatenate(parts + lrows, axis=0)

        CH = 2
        CW = Sq // CH

        def chunk(ref, slot, b, c):
            return ref.at[slot, b, :, pl.ds(c * CW, CW)]

        glob = []
        sends = []
        for b in range(B):
            gb = partial_attn(b)
            glob.append(gb)
            abuf[0, b] = gb.astype(jnp.bfloat16)
            for c in range(CH):
                to_r = rdma(chunk(abuf, 0, b, c), chunk(abuf, 1, b, c),
                            b * CH + c, b * CH + c, right)
                to_l = rdma(chunk(abuf, 0, b, c), chunk(abuf, 2, b, c),
                            4 + b * CH + c, 4 + b * CH + c, left)
                to_r.start()
                to_l.start()
                sends += [to_r, to_l]

        def wait_chunk(slot, b, c, sem, frm):
            rdma(chunk(abuf, 0, b, c), chunk(abuf, slot, b, c),
                 sem, sem, frm).wait_recv()

        for c in range(CH):
            wait_chunk(1, 0, c, c, left)
            f = rdma(chunk(abuf, 1, 0, c), chunk(abuf, 3, 0, c),
                     8 + c, 8 + c, right)
            f.start()
            sends.append(f)

        wait_chunk(2, 0, 0, 4, right)
        wait_chunk(2, 0, 1, 5, right)
        glob[0] = glob[0] + abuf[1, 0].astype(jnp.float32) \
                          + abuf[2, 0].astype(jnp.float32)

        for c in range(CH):
            wait_chunk(2, 1, c, 4 + CH + c, right)
            f = rdma(chunk(abuf, 2, 1, c), chunk(abuf, 3, 1, c),
                     10 + c, 10 + c, left)
            f.start()
            sends.append(f)

        wait_chunk(1, 1, 0, CH, left)
        wait_chunk(1, 1, 1, CH + 1, left)
        glob[1] = glob[1] + abuf[1, 1].astype(jnp.float32) \
                          + abuf[2, 1].astype(jnp.float32)

        wo = wo_ref[...].astype(jnp.bfloat16)

        def finalize(b):
            accT = glob[b] + abuf[3, b].astype(jnp.float32)
            ctx_rows = []
            for h in range(Hq):
                recip = 1.0 / accT[HD + h:HD + h + 1, :]
                ctx_rows.append(accT[h * Dh:(h + 1) * Dh, :] * recip)
            ctxT = jnp.concatenate(ctx_rows, axis=0).astype(jnp.bfloat16)
            out_ref[b] = lax.dot_general(
                ctxT, wo, (((0,), (0,)), ((), ())),
                preferred_element_type=jnp.float32,
            )

        wait_chunk(3, 0, 0, 8, left)
        wait_chunk(3, 0, 1, 9, left)
        finalize(0)
        wait_chunk(3, 1, 0, 10, right)
        wait_chunk(3, 1, 1, 11, right)
        finalize(1)

        for s in sends:
            s.wait_send()

    return pl.pallas_call(
        body,
        out_shape=jax.ShapeDtypeStruct((B, Sq, E), jnp.float32),
        in_specs=[pl.BlockSpec(memory_space=pltpu.VMEM)] * 5,
        out_specs=pl.BlockSpec(memory_space=pltpu.VMEM),
        scratch_shapes=[
            pltpu.VMEM((4, B, ROWS, Sq), jnp.bfloat16),
            pltpu.SemaphoreType.DMA((12,)),
            pltpu.SemaphoreType.DMA((12,)),
        ],
        compiler_params=pltpu.CompilerParams(
            collective_id=0, vmem_limit_bytes=100 * 1024 * 1024
        ),
    )(x, Wq, K2, V2, Wo)


# device time: 41172 ns/iter; 1.2087x vs baseline; 1.0193x over previous
import jax
import jax.numpy as jnp
from jax import lax
from jax.experimental import pallas as pl
from jax.experimental.pallas import tpu as pltpu

N_DEV = 4
BLK = 64
LOG2E = 1.4426950408889634


def kernel(x, Wq, K_ext, V_ext, Wo):
    B, Sq, E = x.shape
    Dm = Wq.shape[1]
    _, Skv_loc, Hq, Dh = K_ext.shape
    HD = Hq * Dh
    ROWS = HD + Hq

    K2 = K_ext.reshape(B, Skv_loc, HD).astype(jnp.bfloat16)
    V2 = V_ext.reshape(B, Skv_loc, HD).astype(jnp.bfloat16)

    def body(x_ref, wq_ref, k_ref, v_ref, wo_ref, out_ref,
             abuf, send_sems, recv_sems):
        my = lax.axis_index("i")
        left = (my + N_DEV - 1) % N_DEV
        right = (my + 1) % N_DEV

        def rdma(src, dst, s_idx, r_idx, dev):
            return pltpu.make_async_remote_copy(
                src_ref=src, dst_ref=dst,
                send_sem=send_sems.at[s_idx], recv_sem=recv_sems.at[r_idx],
                device_id=(dev,), device_id_type=pl.DeviceIdType.MESH,
            )

        barrier_sem = pltpu.get_barrier_semaphore()
        for nbr in (left, right):
            pl.semaphore_signal(
                barrier_sem, inc=1,
                device_id=(nbr,), device_id_type=pl.DeviceIdType.MESH,
            )
        pl.semaphore_wait(barrier_sem, 2)

        wq = wq_ref[...].astype(jnp.bfloat16)
        QTs = []
        for b in range(B):
            q = lax.dot_general(
                x_ref[b].astype(jnp.bfloat16), wq,
                (((1,), (0,)), ((), ())),
                preferred_element_type=jnp.float32,
            )
            q = (q * (0.125 * LOG2E)).astype(jnp.bfloat16)
            QTs.append(q.T)

        kv_blk = lax.broadcasted_iota(jnp.int32, (Skv_loc, Sq), 0) // BLK
        kv_blk = kv_blk + my * (Skv_loc // BLK)
        q_blk = lax.broadcasted_iota(jnp.int32, (Skv_loc, Sq), 1) // BLK
        maskT = (q_blk == kv_blk) | (kv_blk == 0) | ((q_blk + kv_blk) % 3 == 0)
        bias = jnp.where(maskT, 0.0, -1e30).astype(jnp.float32)

        def partial_attn(b, lo, w):
            parts = []
            lrows = []
            for h in range(Hq):
                qhT = QTs[b][h * Dh:(h + 1) * Dh, lo:lo + w]
                kh = k_ref[b, :, h * Dh:(h + 1) * Dh]
                sT = lax.dot_general(
                    kh, qhT, (((1,), (0,)), ((), ())),
                    preferred_element_type=jnp.float32,
                )
                pT = jnp.exp2(sT + bias[:, lo:lo + w])
                lT = pT.sum(axis=0, keepdims=True)
                vh = v_ref[b, :, h * Dh:(h + 1) * Dh]
                accT = lax.dot_general(
                    vh, pT.astype(jnp.bfloat16),
                    (((0,), (0,)), ((), ())),
                    preferred_element_type=jnp.float32,
                )
                parts.append(accT)
                lrows.append(lT)
            return jnp.concatenate(parts + lrows, axis=0)

        CH = 2
        CW = Sq // CH

        def chunk(ref, slot, b, c):
            return ref.at[slot, b, :, pl.ds(c * CW, CW)]

        glob = []
        sends = []
        for b in range(B):
            pieces = []
            for c in range(CH):
                gbc = partial_attn(b, c * CW, CW)
                pieces.append(gbc)
                abuf[0, b, :, c * CW:(c + 1) * CW] = gbc.astype(jnp.bfloat16)
                to_r = rdma(chunk(abuf, 0, b, c), chunk(abuf, 1, b, c),
                            b * CH + c, b * CH + c, right)
                to_l = rdma(chunk(abuf, 0, b, c), chunk(abuf, 2, b, c),
                            4 + b * CH + c, 4 + b * CH + c, left)
                to_r.start()
                to_l.start()
                sends += [to_r, to_l]
            glob.append(jnp.concatenate(pieces, axis=1))

        def wait_chunk(slot, b, c, sem, frm):
            rdma(chunk(abuf, 0, b, c), chunk(abuf, slot, b, c),
                 sem, sem, frm).wait_recv()

        for c in range(CH):
            wait_chunk(1, 0, c, c, left)
            f = rdma(chunk(abuf, 1, 0, c), chunk(abuf, 3, 0, c),
                     8 + c, 8 + c, right)
            f.start()
            sends.append(f)

        wait_chunk(2, 0, 0, 4, right)
        wait_chunk(2, 0, 1, 5, right)
        glob[0] = glob[0] + abuf[1, 0].astype(jnp.float32) \
                          + abuf[2, 0].astype(jnp.float32)

        for c in range(CH):
            wait_chunk(2, 1, c, 4 + CH + c, right)
            f = rdma(chunk(abuf, 2, 1, c), chunk(abuf, 3, 1, c),
                     10 + c, 10 + c, left)
            f.start()
            sends.append(f)

        wait_chunk(1, 1, 0, CH, left)
        wait_chunk(1, 1, 1, CH + 1, left)
        glob[1] = glob[1] + abuf[1, 1].astype(jnp.float32) \
                          + abuf[2, 1].astype(jnp.float32)

        wo = wo_ref[...].astype(jnp.bfloat16)

        def finalize(b):
            accT = glob[b] + abuf[3, b].astype(jnp.float32)
            ctx_rows = []
            for h in range(Hq):
                recip = 1.0 / accT[HD + h:HD + h + 1, :]
                ctx_rows.append(accT[h * Dh:(h + 1) * Dh, :] * recip)
            ctxT = jnp.concatenate(ctx_rows, axis=0).astype(jnp.bfloat16)
            out_ref[b] = lax.dot_general(
                ctxT, wo, (((0,), (0,)), ((), ())),
                preferred_element_type=jnp.float32,
            )

        wait_chunk(3, 0, 0, 8, left)
        wait_chunk(3, 0, 1, 9, left)
        finalize(0)
        wait_chunk(3, 1, 0, 10, right)
        wait_chunk(3, 1, 1, 11, right)
        finalize(1)

        for s in sends:
            s.wait_send()

    return pl.pallas_call(
        body,
        out_shape=jax.ShapeDtypeStruct((B, Sq, E), jnp.float32),
        in_specs=[pl.BlockSpec(memory_space=pltpu.VMEM)] * 5,
        out_specs=pl.BlockSpec(memory_space=pltpu.VMEM),
        scratch_shapes=[
            pltpu.VMEM((4, B, ROWS, Sq), jnp.bfloat16),
            pltpu.SemaphoreType.DMA((12,)),
            pltpu.SemaphoreType.DMA((12,)),
        ],
        compiler_params=pltpu.CompilerParams(
            collective_id=0, vmem_limit_bytes=100 * 1024 * 1024
        ),
    )(x, Wq, K2, V2, Wo)
